# baseline (speedup 1.0000x reference)
import functools

import jax
import jax.numpy as jnp
import numpy as np
from jax import lax
from jax.experimental import pallas as pl
from jax.experimental.pallas import tpu as pltpu
from jax.experimental.pallas import tpu_sc as plsc

_IMG_SIZE = 224
_CENTERS_LEFT = [4, 1, 2, 24, 19, 16, 31, 31, 34, 34, 37, 43]
_CENTERS_RIGHT = [5, 8, 7, 29, 24, 16, 37, 37, 34, 34, 45, 47]
_LOC_SCALE = [0.5, 0.33, -0.5, 0.25, 0.0, 0.16, -0.16, 0.3, 0.0, -0.3, 0.5, -0.25]

_B, _C, _H, _W = 32, 256, 56, 56
_HW = _H * _W
_SCALE = min(_H, _W) / _IMG_SIZE
_NCTR = 12
_NLM = 49

_RNE_BIAS = np.float32(1.5 * 2**23)


def _rne(v):
    return (v + _RNE_BIAS) - _RNE_BIAS


_mesh = plsc.VectorSubcoreMesh(core_axis_name="c", subcore_axis_name="s",
                               num_cores=2)


@functools.partial(
    pl.kernel,
    mesh=_mesh,
    out_type=[
        jax.ShapeDtypeStruct((_NCTR, _B, _C), jnp.float32),
        jax.ShapeDtypeStruct((_NCTR, _B, _C), jnp.float32),
    ],
    scratch_types=[
        pltpu.VMEM((_B, 2, _NLM), jnp.float32),
        pltpu.VMEM((32,), jnp.int32),
        pltpu.VMEM((_B, _C), jnp.float32),
        pltpu.SemaphoreType.DMA,
        pltpu.SemaphoreType.DMA,
    ],
    compiler_params=pltpu.CompilerParams(needs_layout_passes=False, skip_device_barrier=True, disable_bounds_checks=True, disable_semaphore_checks=True),
)
def _sc_gather(rows_hbm, lm_hbm, enc_hbm, dec_hbm, lm_v, rid_v, buf_v,
               lm_sem, sem):
    w = lax.axis_index("s") * 2 + lax.axis_index("c")

    @pl.when(w < 2 * _NCTR)
    def _body():
        lm_copy = pltpu.async_copy(lm_hbm, lm_v, lm_sem)

        is_right = (w >= _NCTR).astype(jnp.int32)
        j = w - _NCTR * is_right
        cid_s = jnp.int32(0)
        lsj_s = jnp.float32(0)
        for i in range(_NCTR):
            m = (j == i).astype(jnp.int32)
            cid_s = cid_s + m * (_CENTERS_LEFT[i]
                                 + is_right * (_CENTERS_RIGHT[i] - _CENTERS_LEFT[i]))
            lsj_s = lsj_s + m.astype(jnp.float32) * _LOC_SCALE[i]
        cid = jnp.full((16,), cid_s, jnp.int32)
        lsj = jnp.full((16,), lsj_s, jnp.float32)

        lm_copy.wait()
        lanes = lax.iota(jnp.int32, 16)
        zero = jnp.zeros((16,), jnp.int32)
        one = jnp.full((16,), 1, jnp.int32)
        for k in range(2):
            b_vec = k * 16 + lanes
            xr = plsc.load_gather(lm_v, [b_vec, zero, jnp.full((16,), 22, jnp.int32)])
            yr = plsc.load_gather(lm_v, [b_vec, zero, jnp.full((16,), 25, jnp.int32)])
            scale = jnp.abs(xr - yr) * lsj
            x = plsc.load_gather(lm_v, [b_vec, zero, cid])
            y = plsc.load_gather(lm_v, [b_vec, one, cid]) + scale
            xi = _rne(_rne(x) * _SCALE).astype(jnp.int32)
            yi = _rne(_rne(y) * _SCALE).astype(jnp.int32)
            xi = jnp.clip(xi, 0, _W - 1)
            yi = jnp.clip(yi, 0, _H - 1)
            rid_v[pl.ds(k * 16, 16)] = b_vec * _HW + xi * _W + yi

        g1 = pltpu.async_copy(rows_hbm.at[rid_v.at[pl.ds(0, 16)]],
                              buf_v.at[pl.ds(0, 16)], sem)
        g2 = pltpu.async_copy(rows_hbm.at[rid_v.at[pl.ds(16, 16)]],
                              buf_v.at[pl.ds(16, 16)], lm_sem)
        g1.wait()

        @pl.when(is_right == 0)
        def _enc():
            o1 = pltpu.async_copy(buf_v.at[pl.ds(0, 16)],
                                  enc_hbm.at[j, pl.ds(0, 16)], sem)
            g2.wait()
            o2 = pltpu.async_copy(buf_v.at[pl.ds(16, 16)],
                                  enc_hbm.at[j, pl.ds(16, 16)], sem)
            o1.wait()
            o2.wait()

        @pl.when(is_right == 1)
        def _dec():
            o1 = pltpu.async_copy(buf_v.at[pl.ds(0, 16)],
                                  dec_hbm.at[j, pl.ds(0, 16)], sem)
            g2.wait()
            o2 = pltpu.async_copy(buf_v.at[pl.ds(16, 16)],
                                  dec_hbm.at[j, pl.ds(16, 16)], sem)
            o1.wait()
            o2.wait()


def kernel(tensor, landmarks):
    batch, channels, h, w = tensor.shape
    rows = tensor.transpose(0, 2, 3, 1).reshape(batch * h * w, channels)
    enc, dec = _sc_gather(rows, landmarks)
    return (enc.transpose(1, 0, 2), dec.transpose(1, 0, 2))


# speedup vs baseline: 1.0028x; 1.0028x over previous
import functools

import jax
import jax.numpy as jnp
import numpy as np
from jax import lax
from jax.experimental import pallas as pl
from jax.experimental.pallas import tpu as pltpu
from jax.experimental.pallas import tpu_sc as plsc

_IMG_SIZE = 224
_CENTERS_LEFT = [4, 1, 2, 24, 19, 16, 31, 31, 34, 34, 37, 43]
_CENTERS_RIGHT = [5, 8, 7, 29, 24, 16, 37, 37, 34, 34, 45, 47]
_LOC_SCALE = [0.5, 0.33, -0.5, 0.25, 0.0, 0.16, -0.16, 0.3, 0.0, -0.3, 0.5, -0.25]

_B, _C, _H, _W = 32, 256, 56, 56
_HW = _H * _W
_SCALE = min(_H, _W) / _IMG_SIZE
_NCTR = 12
_NLM = 49

_RNE_BIAS = np.float32(1.5 * 2**23)


def _rne(v):
    return (v + _RNE_BIAS) - _RNE_BIAS


_mesh = plsc.VectorSubcoreMesh(core_axis_name="c", subcore_axis_name="s",
                               num_cores=2)


@functools.partial(
    pl.kernel,
    mesh=_mesh,
    out_type=[
        jax.ShapeDtypeStruct((_NCTR, _B, _C), jnp.float32),
        jax.ShapeDtypeStruct((_NCTR, _B, _C), jnp.float32),
    ],
    scratch_types=[
        pltpu.VMEM((_B, 2, _NLM), jnp.float32),
        pltpu.VMEM((32,), jnp.int32),
        pltpu.VMEM((_B, _C), jnp.float32),
        pltpu.SemaphoreType.DMA,
        pltpu.SemaphoreType.DMA,
    ],
    compiler_params=pltpu.CompilerParams(needs_layout_passes=False, skip_device_barrier=True, disable_bounds_checks=True, disable_semaphore_checks=True, internal_scratch_in_bytes=1048576),
)
def _sc_gather(rows_hbm, lm_hbm, enc_hbm, dec_hbm, lm_v, rid_v, buf_v,
               lm_sem, sem):
    w = lax.axis_index("s") * 2 + lax.axis_index("c")

    @pl.when(w < 2 * _NCTR)
    def _body():
        lm_copy = pltpu.async_copy(lm_hbm, lm_v, lm_sem)

        is_right = (w >= _NCTR).astype(jnp.int32)
        j = w - _NCTR * is_right
        cid_s = jnp.int32(0)
        lsj_s = jnp.float32(0)
        for i in range(_NCTR):
            m = (j == i).astype(jnp.int32)
            cid_s = cid_s + m * (_CENTERS_LEFT[i]
                                 + is_right * (_CENTERS_RIGHT[i] - _CENTERS_LEFT[i]))
            lsj_s = lsj_s + m.astype(jnp.float32) * _LOC_SCALE[i]
        cid = jnp.full((16,), cid_s, jnp.int32)
        lsj = jnp.full((16,), lsj_s, jnp.float32)

        lm_copy.wait()
        lanes = lax.iota(jnp.int32, 16)
        zero = jnp.zeros((16,), jnp.int32)
        one = jnp.full((16,), 1, jnp.int32)
        for k in range(2):
            b_vec = k * 16 + lanes
            xr = plsc.load_gather(lm_v, [b_vec, zero, jnp.full((16,), 22, jnp.int32)])
            yr = plsc.load_gather(lm_v, [b_vec, zero, jnp.full((16,), 25, jnp.int32)])
            scale = jnp.abs(xr - yr) * lsj
            x = plsc.load_gather(lm_v, [b_vec, zero, cid])
            y = plsc.load_gather(lm_v, [b_vec, one, cid]) + scale
            xi = _rne(_rne(x) * _SCALE).astype(jnp.int32)
            yi = _rne(_rne(y) * _SCALE).astype(jnp.int32)
            xi = jnp.clip(xi, 0, _W - 1)
            yi = jnp.clip(yi, 0, _H - 1)
            rid_v[pl.ds(k * 16, 16)] = b_vec * _HW + xi * _W + yi

        g1 = pltpu.async_copy(rows_hbm.at[rid_v.at[pl.ds(0, 16)]],
                              buf_v.at[pl.ds(0, 16)], sem)
        g2 = pltpu.async_copy(rows_hbm.at[rid_v.at[pl.ds(16, 16)]],
                              buf_v.at[pl.ds(16, 16)], lm_sem)
        g1.wait()

        @pl.when(is_right == 0)
        def _enc():
            o1 = pltpu.async_copy(buf_v.at[pl.ds(0, 16)],
                                  enc_hbm.at[j, pl.ds(0, 16)], sem)
            g2.wait()
            o2 = pltpu.async_copy(buf_v.at[pl.ds(16, 16)],
                                  enc_hbm.at[j, pl.ds(16, 16)], sem)
            o1.wait()
            o2.wait()

        @pl.when(is_right == 1)
        def _dec():
            o1 = pltpu.async_copy(buf_v.at[pl.ds(0, 16)],
                                  dec_hbm.at[j, pl.ds(0, 16)], sem)
            g2.wait()
            o2 = pltpu.async_copy(buf_v.at[pl.ds(16, 16)],
                                  dec_hbm.at[j, pl.ds(16, 16)], sem)
            o1.wait()
            o2.wait()


def kernel(tensor, landmarks):
    batch, channels, h, w = tensor.shape
    rows = tensor.transpose(0, 2, 3, 1).reshape(batch * h * w, channels)
    enc, dec = _sc_gather(rows, landmarks)
    return (enc.transpose(1, 0, 2), dec.transpose(1, 0, 2))
